# Initial kernel scaffold; baseline (speedup 1.0000x reference)
import functools

import jax
import jax.numpy as jnp
from jax import lax
from jax.experimental import pallas as pl
from jax.experimental.pallas import tpu as pltpu
from jax.experimental.pallas import tpu_sc as plsc

CYCLE = 168
LEN = 336
D = 128
B = 1024

NC = 2
NS = 16
LANES = 16
NW = NC * NS
BPW = B // NW
VPS = LEN // LANES
NVEC = BPW * VPS


def _sc_cycle_gather(index, data):
  mesh = plsc.VectorSubcoreMesh(core_axis_name="c", subcore_axis_name="s")

  @functools.partial(
      pl.kernel,
      out_type=jax.ShapeDtypeStruct((B * LEN, D), jnp.float32),
      mesh=mesh,
      scratch_types=[
          pltpu.VMEM((BPW,), jnp.int32),
          pltpu.VMEM((BPW * LEN,), jnp.int32),
          pltpu.VMEM((2, LEN, D), jnp.float32),
          pltpu.SemaphoreType.DMA,
          pltpu.SemaphoreType.DMA,
      ],
  )
  def k(index_hbm, data_hbm, out_hbm, sidx_v, ridx_v, rows_v, gsem, wsem):
    wid = lax.axis_index("s") * NC + lax.axis_index("c")
    base = wid * BPW
    pltpu.sync_copy(index_hbm.at[pl.ds(base, BPW)], sidx_v)

    iota = lax.iota(jnp.int32, LANES)

    def expand(v, carry):
      s = v // VPS
      kk = v % VPS
      bcast = plsc.load_gather(sidx_v, [jnp.full((LANES,), s, jnp.int32)])
      row = lax.rem(bcast + iota + kk * LANES, CYCLE)
      ridx_v[pl.ds(v * LANES, LANES)] = row
      return carry

    lax.fori_loop(0, NVEC, expand, 0)

    def body(s, carry):
      pltpu.async_copy(
          data_hbm.at[ridx_v.at[pl.ds(s * LEN, LEN)]], rows_v.at[0],
          gsem).wait()
      pltpu.async_copy(
          rows_v.at[0], out_hbm.at[pl.ds((base + s) * LEN, LEN)],
          wsem).wait()
      return carry

    lax.fori_loop(0, BPW, body, 0)

  return k(index, data)


def kernel(index, length, data):
  del length
  out = _sc_cycle_gather(index.astype(jnp.int32), data)
  return out.reshape(B, LEN, D)


# speedup vs baseline: 14.2590x; 14.2590x over previous
import functools

import jax
import jax.numpy as jnp
from jax import lax
from jax.experimental import pallas as pl
from jax.experimental.pallas import tpu as pltpu
from jax.experimental.pallas import tpu_sc as plsc

CYCLE = 168
LEN = 336
D = 128
B = 1024

NC = 2
NS = 16
NW = NC * NS
BPW = B // NW


def _sc_cycle_gather(index, data):
  mesh = plsc.VectorSubcoreMesh(core_axis_name="c", subcore_axis_name="s")

  @functools.partial(
      pl.kernel,
      out_type=jax.ShapeDtypeStruct((B * LEN, D), jnp.float32),
      mesh=mesh,
      scratch_types=[
          pltpu.VMEM((BPW + 16,), jnp.int32),
          pltpu.VMEM((3 * CYCLE, D), jnp.float32),
          pltpu.SemaphoreType.DMA,
          pltpu.SemaphoreType.DMA,
      ],
  )
  def k(index_hbm, data_hbm, out_hbm, sidx_v, d3_v, tsem, wsem):
    wid = lax.axis_index("s") * NC + lax.axis_index("c")
    base = wid * BPW

    cp0 = pltpu.async_copy(data_hbm, d3_v.at[pl.ds(0, CYCLE)], tsem)
    cp1 = pltpu.async_copy(data_hbm, d3_v.at[pl.ds(CYCLE, CYCLE)], tsem)
    cp2 = pltpu.async_copy(data_hbm, d3_v.at[pl.ds(2 * CYCLE, CYCLE)], tsem)
    pltpu.sync_copy(index_hbm.at[pl.ds(base, BPW)], sidx_v.at[pl.ds(0, BPW)])
    cp0.wait()
    cp1.wait()
    cp2.wait()

    def body(s, carry):
      r = sidx_v[pl.ds(s, 16)][0]
      pltpu.async_copy(
          d3_v.at[pl.ds(r, LEN)], out_hbm.at[pl.ds((base + s) * LEN, LEN)],
          wsem).wait()
      return carry

    lax.fori_loop(0, BPW, body, 0)

  return k(index, data)


def kernel(index, length, data):
  del length
  out = _sc_cycle_gather(index.astype(jnp.int32), data)
  return out.reshape(B, LEN, D)
